# Initial kernel scaffold; baseline (speedup 1.0000x reference)
import jax
import jax.numpy as jnp
from jax.experimental import pallas as pl


def kernel(neighbor_user_ids, neighbor_mask, emb_table, proj_W, proj_b, ln_g, ln_b, att_W1, att_b1, att_w2, att_b2):
    raise NotImplementedError("write your pallas kernel here")


# speedup vs baseline: 5.3973x; 5.3973x over previous
import functools

import jax
import jax.numpy as jnp
from jax import lax
from jax.experimental import pallas as pl
from jax.experimental.pallas import tpu as pltpu
from jax.experimental.pallas import tpu_sc as plsc

_CH = 128


def _sc_gather(emb_table, idx3, n_rows):
    nw, n_ch, ch = idx3.shape
    d = emb_table.shape[1]
    per_w = n_ch * ch
    np_half = n_ch // 2

    mesh = plsc.VectorSubcoreMesh(core_axis_name="c", subcore_axis_name="s")
    nc = 2

    @functools.partial(
        pl.kernel,
        mesh=mesh,
        compiler_params=pltpu.CompilerParams(use_tc_tiling_on_sc=False),
        out_type=jax.ShapeDtypeStruct((n_rows, d), jnp.float32),
        scratch_types=[
            pltpu.VMEM((n_ch, ch), jnp.int32),
            pltpu.VMEM((2, ch, d), jnp.float32),
            pltpu.SemaphoreType.DMA,
            pltpu.SemaphoreType.DMA,
            pltpu.SemaphoreType.DMA,
            pltpu.SemaphoreType.DMA,
        ],
    )
    def k(table_hbm, idx_hbm, out_hbm, idx_v, rows_v, g0, g1, o0, o1):
        wid = lax.axis_index("s") * nc + lax.axis_index("c")
        base = wid * per_w
        pltpu.sync_copy(idx_hbm.at[wid], idx_v)

        def start_gather(j, b, sem):
            pltpu.async_copy(table_hbm.at[idx_v.at[j]], rows_v.at[b], sem)

        def wait_gather(j, b, sem):
            pltpu.make_async_copy(
                table_hbm.at[idx_v.at[j]], rows_v.at[b], sem
            ).wait()

        def start_out(j, b, sem):
            pltpu.async_copy(
                rows_v.at[b], out_hbm.at[pl.ds(base + j * ch, ch)], sem
            )

        def wait_out(j, b, sem):
            pltpu.make_async_copy(
                rows_v.at[b], out_hbm.at[pl.ds(base + j * ch, ch)], sem
            ).wait()

        start_gather(0, 0, g0)
        start_gather(1, 1, g1)

        def body(p, carry):
            j0 = 2 * p
            wait_gather(j0, 0, g0)
            start_out(j0, 0, o0)
            wait_gather(j0 + 1, 1, g1)
            start_out(j0 + 1, 1, o1)
            wait_out(j0, 0, o0)
            start_gather(j0 + 2, 0, g0)
            wait_out(j0 + 1, 1, o1)
            start_gather(j0 + 3, 1, g1)
            return carry

        lax.fori_loop(0, np_half - 1, body, 0)

        jl = 2 * (np_half - 1)
        wait_gather(jl, 0, g0)
        start_out(jl, 0, o0)
        wait_gather(jl + 1, 1, g1)
        start_out(jl + 1, 1, o1)
        wait_out(jl, 0, o0)
        wait_out(jl + 1, 1, o1)

    return k(emb_table, idx3)


def _tc_dense(g_flat, mask2, proj_W, proj_b, ln_g, ln_b, att_W1, att_b1,
              att_w2r, B, N, H, GB):
    M = GB * N

    def body(g_ref, m_ref, W_ref, pb_ref, lg_ref, lb_ref, W1_ref, b1_ref,
             w2_ref, out_ref, attn_ref):
        x = g_ref[...]
        v = jnp.dot(x, W_ref[...], preferred_element_type=jnp.float32)
        v = v + pb_ref[...]
        v = jnp.maximum(v, 0.0)
        mu = jnp.mean(v, axis=1, keepdims=True)
        dvec = v - mu
        var = jnp.mean(dvec * dvec, axis=1, keepdims=True)
        v = dvec * lax.rsqrt(var + 1e-5) * lg_ref[...] + lb_ref[...]
        h = jnp.tanh(
            jnp.dot(v, W1_ref[...], preferred_element_type=jnp.float32)
            + b1_ref[...]
        )
        s = jnp.sum(h * w2_ref[...], axis=1, keepdims=True)
        s3 = s.reshape(GB, N, 1)
        m3 = m_ref[...].reshape(GB, N, 1)
        s3 = jnp.where(m3 != 0, s3, jnp.float32(-1e9))
        smax = jnp.max(s3, axis=1, keepdims=True)
        e3 = jnp.exp(s3 - smax)
        denom = jnp.sum(e3, axis=1, keepdims=True)
        w3 = e3 / denom
        attn_ref[...] = w3.reshape(M, 1)
        v3 = v.reshape(GB, N, H)
        out_ref[...] = jnp.sum(w3 * v3, axis=1)

    D = g_flat.shape[1]
    grid = (B // GB,)
    out, attn = pl.pallas_call(
        body,
        grid=grid,
        in_specs=[
            pl.BlockSpec((M, D), lambda i: (i, 0)),
            pl.BlockSpec((M, 1), lambda i: (i, 0)),
            pl.BlockSpec((D, H), lambda i: (0, 0)),
            pl.BlockSpec((1, H), lambda i: (0, 0)),
            pl.BlockSpec((1, H), lambda i: (0, 0)),
            pl.BlockSpec((1, H), lambda i: (0, 0)),
            pl.BlockSpec((H, H), lambda i: (0, 0)),
            pl.BlockSpec((1, H), lambda i: (0, 0)),
            pl.BlockSpec((1, H), lambda i: (0, 0)),
        ],
        out_specs=(
            pl.BlockSpec((GB, H), lambda i: (i, 0)),
            pl.BlockSpec((M, 1), lambda i: (i, 0)),
        ),
        out_shape=(
            jax.ShapeDtypeStruct((B, H), jnp.float32),
            jax.ShapeDtypeStruct((B * N, 1), jnp.float32),
        ),
    )(g_flat, mask2, proj_W, proj_b, ln_g, ln_b, att_W1, att_b1, att_w2r)
    return out, attn


def kernel(neighbor_user_ids, neighbor_mask, emb_table, proj_W, proj_b,
           ln_g, ln_b, att_W1, att_b1, att_w2, att_b2):
    B, N = neighbor_user_ids.shape
    D = emb_table.shape[1]
    H = att_W1.shape[0]

    info = plsc.get_sparse_core_info()
    nw = info.num_cores * info.num_subcores
    total = B * N
    assert total % (nw * _CH) == 0
    idx3 = neighbor_user_ids.reshape(-1).astype(jnp.int32).reshape(nw, -1, _CH)

    gathered = _sc_gather(emb_table, idx3, total)

    mask_col = neighbor_mask.reshape(total, 1).astype(jnp.int32)
    GB = 128
    out, attn = _tc_dense(
        gathered, mask_col, proj_W, proj_b.reshape(1, H), ln_g.reshape(1, H),
        ln_b.reshape(1, H), att_W1, att_b1.reshape(1, H),
        att_w2.reshape(1, H), B, N, H, GB,
    )
    return out, attn.reshape(B, N)


# speedup vs baseline: 6.5000x; 1.2043x over previous
import functools

import jax
import jax.numpy as jnp
from jax import lax
from jax.experimental import pallas as pl
from jax.experimental.pallas import tpu as pltpu
from jax.experimental.pallas import tpu_sc as plsc

_CH = 128


def _sc_gather(table4, idx3, n_rows, d):
    nw, n_ch, ch = idx3.shape
    per_w = n_ch * ch
    np_half = n_ch // 2

    mesh = plsc.VectorSubcoreMesh(core_axis_name="c", subcore_axis_name="s")
    nc = 2

    @functools.partial(
        pl.kernel,
        mesh=mesh,
        out_type=jax.ShapeDtypeStruct((n_rows, d), jnp.float32),
        scratch_types=[
            pltpu.VMEM((n_ch, ch), jnp.int32),
            pltpu.VMEM((n_ch, ch), jnp.int32),
            pltpu.VMEM((2, ch, 128), jnp.float32),
            pltpu.VMEM((2, ch, d), jnp.float32),
            pltpu.SemaphoreType.DMA,
            pltpu.SemaphoreType.DMA,
            pltpu.SemaphoreType.DMA,
            pltpu.SemaphoreType.DMA,
        ],
    )
    def k(t4_hbm, idx_hbm, out_hbm, idx_v, rows_v, slab_v, pack_v,
          g0, g1, o0, o1):
        wid = lax.axis_index("s") * nc + lax.axis_index("c")
        base = wid * per_w
        pltpu.sync_copy(idx_hbm.at[wid], idx_v)

        def rowcalc(j, carry):
            for kk in range(ch // 16):
                sl = pl.ds(kk * 16, 16)
                rows_v[j, sl] = lax.shift_right_logical(idx_v[j, sl], 2)
            return carry

        lax.fori_loop(0, n_ch, rowcalc, 0)

        def start_gather(j, b, sem):
            pltpu.async_copy(t4_hbm.at[rows_v.at[j]], slab_v.at[b], sem)

        def wait_gather(j, b, sem):
            pltpu.make_async_copy(
                t4_hbm.at[rows_v.at[j]], slab_v.at[b], sem
            ).wait()

        def extract(j, b):
            def body(g, carry):
                idvec = idx_v[j, pl.ds(g * 16, 16)]
                offs = (idvec & 3) * d
                for l in range(16):
                    i = g * 16 + l
                    off = offs[l]
                    for kk in range(d // 16):
                        pack_v[b, i, pl.ds(kk * 16, 16)] = (
                            slab_v[b, i, pl.ds(off + kk * 16, 16)]
                        )
                return carry

            lax.fori_loop(0, ch // 16, body, 0)

        def start_out(j, b, sem):
            pltpu.async_copy(
                pack_v.at[b], out_hbm.at[pl.ds(base + j * ch, ch)], sem
            )

        def wait_out(j, b, sem):
            pltpu.make_async_copy(
                pack_v.at[b], out_hbm.at[pl.ds(base + j * ch, ch)], sem
            ).wait()

        start_gather(0, 0, g0)
        start_gather(1, 1, g1)

        def body(p, carry):
            j0 = 2 * p

            def half(j, b, gs, os):
                wait_gather(j, b, gs)

                @pl.when(p > 0)
                def _():
                    wait_out(j - 2, b, os)

                extract(j, b)
                start_out(j, b, os)

                @pl.when(p < np_half - 1)
                def _():
                    start_gather(j + 2, b, gs)

            half(j0, 0, g0, o0)
            half(j0 + 1, 1, g1, o1)
            return carry

        lax.fori_loop(0, np_half, body, 0)

        jl = 2 * (np_half - 1)
        wait_out(jl, 0, o0)
        wait_out(jl + 1, 1, o1)

    return k(table4, idx3)


def _tc_dense(g_flat, mask2, fold, foldT, proj_W, proj_b, ln_g, ln_b,
              att_W1, att_b1, att_w2, B, N, H, GB):
    M = GB * N

    def body(g_ref, m_ref, q_ref, qt_ref, W_ref, pb_ref, lg_ref, lb_ref,
             W1_ref, b1_ref, w2_ref, out_ref, attn_ref):
        x = g_ref[...]
        v = jnp.dot(x, W_ref[...], preferred_element_type=jnp.float32)
        v = jnp.maximum(v + pb_ref[...], 0.0)
        ones_col = jnp.full((H, 1), 1.0 / H, dtype=jnp.float32)
        mu = jnp.dot(v, ones_col, preferred_element_type=jnp.float32)
        m2 = jnp.dot(v * v, ones_col, preferred_element_type=jnp.float32)
        var = jnp.maximum(m2 - mu * mu, 0.0)
        v = (v - mu) * lax.rsqrt(var + 1e-5) * lg_ref[...] + lb_ref[...]
        h = jnp.tanh(
            jnp.dot(v, W1_ref[...], preferred_element_type=jnp.float32)
            + b1_ref[...]
        )
        s = jnp.dot(h, w2_ref[...], preferred_element_type=jnp.float32)
        e = jnp.where(m_ref[...] != 0, jnp.exp(s), 0.0)
        denom_b = jnp.dot(
            q_ref[...], e, preferred_element_type=jnp.float32
        )
        denom_col = jnp.dot(
            qt_ref[...], denom_b, preferred_element_type=jnp.float32
        )
        w_col = e / denom_col
        attn_ref[...] = w_col
        out_ref[...] = jnp.dot(
            q_ref[...], w_col * v, preferred_element_type=jnp.float32
        )

    D = g_flat.shape[1]
    grid = (B // GB,)
    out, attn = pl.pallas_call(
        body,
        grid=grid,
        in_specs=[
            pl.BlockSpec((M, D), lambda i: (i, 0)),
            pl.BlockSpec((M, 1), lambda i: (i, 0)),
            pl.BlockSpec((GB, M), lambda i: (0, 0)),
            pl.BlockSpec((M, GB), lambda i: (0, 0)),
            pl.BlockSpec((D, H), lambda i: (0, 0)),
            pl.BlockSpec((1, H), lambda i: (0, 0)),
            pl.BlockSpec((1, H), lambda i: (0, 0)),
            pl.BlockSpec((1, H), lambda i: (0, 0)),
            pl.BlockSpec((H, H), lambda i: (0, 0)),
            pl.BlockSpec((1, H), lambda i: (0, 0)),
            pl.BlockSpec((H, 1), lambda i: (0, 0)),
        ],
        out_specs=(
            pl.BlockSpec((GB, H), lambda i: (i, 0)),
            pl.BlockSpec((M, 1), lambda i: (i, 0)),
        ),
        out_shape=(
            jax.ShapeDtypeStruct((B, H), jnp.float32),
            jax.ShapeDtypeStruct((B * N, 1), jnp.float32),
        ),
    )(g_flat, mask2, fold, foldT, proj_W, proj_b, ln_g, ln_b, att_W1,
      att_b1, att_w2)
    return out, attn


def kernel(neighbor_user_ids, neighbor_mask, emb_table, proj_W, proj_b,
           ln_g, ln_b, att_W1, att_b1, att_w2, att_b2):
    B, N = neighbor_user_ids.shape
    D = emb_table.shape[1]
    H = att_W1.shape[0]

    info = plsc.get_sparse_core_info()
    nw = info.num_cores * info.num_subcores
    total = B * N
    assert total % (nw * _CH) == 0 and (emb_table.shape[0] * D) % 128 == 0
    idx3 = neighbor_user_ids.reshape(-1).astype(jnp.int32).reshape(nw, -1, _CH)
    flat = lax.optimization_barrier(emb_table.reshape(-1))
    table4 = flat.reshape(emb_table.shape[0] * D // 128, 128)

    gathered = _sc_gather(table4, idx3, total, D)

    mask_col = neighbor_mask.reshape(total, 1).astype(jnp.int32)
    GB = 128
    M = GB * N
    fold = (jax.lax.broadcasted_iota(jnp.int32, (GB, M), 1) // N
            == jax.lax.broadcasted_iota(jnp.int32, (GB, M), 0)
            ).astype(jnp.float32)
    out, attn = _tc_dense(
        gathered, mask_col, fold, fold.T, proj_W, proj_b.reshape(1, H),
        ln_g.reshape(1, H), ln_b.reshape(1, H), att_W1, att_b1.reshape(1, H),
        att_w2, B, N, H, GB,
    )
    return out, attn.reshape(B, N)
